# baseline (speedup 1.0000x reference)
import functools

import jax
import jax.numpy as jnp
import numpy as np
from jax import lax
from jax.experimental import pallas as pl
from jax.experimental.pallas import tpu as pltpu
from jax.experimental.pallas import tpu_sc as plsc

_N = 10000
_E = 160000

_NC, _NS = 2, 16
_NW = _NC * _NS
_GC = 128
_W = 128
_G8 = 1024
_NGF = _E // _G8
_JF = _NGF // _NW
_GEXT = _NGF - _JF * _NW
_REMB = _NGF * _G8
_REMC = _E - _REMB

_EB = 800
_K = 17 * 128

_S3 = 1.0 / np.sqrt(3.0)
_INV = 1.0 / np.sqrt(32.0)

_PERM = np.concatenate([np.arange(16)] + [16 + 3 * np.arange(16) + k for k in range(3)])
_COLMAP = np.zeros(64, dtype=np.int32)
_COLMAP[:16] = np.arange(16)
for _w in range(16):
    for _k in range(3):
        _COLMAP[16 + 3 * _w + _k] = 16 + 16 * _k + _w


def _assemble_wc(W2, b2):
    W2r = W2.reshape(16, 4, 16, 16)
    b2r = b2.reshape(4, 16, 16)
    T = jnp.zeros((17, 128, 128), jnp.float32)
    T = T.at[:16, 0:16, 0:16].set(W2r[:, 0])
    T = T.at[:16, 16:32, 0:16].set(_S3 * W2r[:, 3])
    T = T.at[:16, 32:48, 16:32].set(W2r[:, 1])
    for kk in range(3):
        T = T.at[:16, 48 + 16 * kk:64 + 16 * kk, 32 + 16 * kk:48 + 16 * kk].set(W2r[:, 2])
    T = T.at[16, 0:16, 0:16].set(b2r[0])
    T = T.at[16, 16:32, 0:16].set(_S3 * b2r[3])
    T = T.at[16, 32:48, 16:32].set(b2r[1])
    for kk in range(3):
        T = T.at[16, 48 + 16 * kk:64 + 16 * kk, 32 + 16 * kk:48 + 16 * kk].set(b2r[2])
    return T.transpose(1, 0, 2).reshape(128, 17 * 128)


_REP = np.zeros((16, 16 * 128), np.float32)
for _kk in range(16):
    _REP[_kk, 128 * _kk:128 * (_kk + 1)] = 1.0
_SHB = np.zeros((4, 64), np.float32)
for _kk in range(4):
    _SHB[_kk, 16 * _kk:16 * (_kk + 1)] = 1.0


def _sc_gather(na, src):
    mesh = plsc.VectorSubcoreMesh(core_axis_name="c", subcore_axis_name="s")

    @functools.partial(
        pl.kernel,
        out_type=jax.ShapeDtypeStruct((_E, _W), jnp.float32),
        mesh=mesh,
        scratch_types=[
            pltpu.VMEM((_G8,), jnp.int32),
            pltpu.VMEM((512, _W), jnp.float32),
            pltpu.SemaphoreType.DMA,
        ],
    )
    def gk(na_hbm, src_hbm, x_hbm, idx_v, rows_v, sem):
        c = lax.axis_index("c")
        s = lax.axis_index("s")
        wid = s * _NC + c

        def group(gid):
            base = pl.multiple_of(gid * _G8, _G8)
            pltpu.sync_copy(src_hbm.at[pl.ds(base, _G8)], idx_v)
            for half in range(2):
                descs = [pltpu.async_copy(
                    na_hbm.at[idx_v.at[pl.ds(half * 512 + _GC * i, _GC)]],
                    rows_v.at[pl.ds(_GC * i, _GC)], sem) for i in range(4)]
                for d in descs:
                    d.wait()
                pltpu.sync_copy(rows_v, x_hbm.at[pl.ds(base + half * 512, 512)])

        def body(j, carry):
            group(j * _NW + wid)
            return carry

        lax.fori_loop(0, _JF, body, 0)

        @pl.when(wid < _GEXT)
        def _():
            group(_JF * _NW + wid)

        @pl.when(wid == _GEXT)
        def _():
            pltpu.sync_copy(src_hbm.at[pl.ds(_REMB, _REMC)], idx_v.at[pl.ds(0, _REMC)])
            descs = [pltpu.async_copy(
                na_hbm.at[idx_v.at[pl.ds(_GC * i, _GC)]],
                rows_v.at[pl.ds(_GC * i, _GC)], sem) for i in range(_REMC // _GC)]
            for d in descs:
                d.wait()
            pltpu.sync_copy(rows_v.at[pl.ds(0, _REMC)], x_hbm.at[pl.ds(_REMB, _REMC)])

    return gk(na, src)


def _tc_dense(x, ea, sh, W1, b1, Wc, Rep, Shb):
    def body(x_ref, ea_ref, sh_ref, w1_ref, b1_ref, wc_ref, rep_ref,
             shb_ref, o_ref):
        xb = x_ref[...]
        a = jnp.dot(ea_ref[...], w1_ref[...],
                    preferred_element_type=jnp.float32) + b1_ref[...]
        h = a * (1.0 / (1.0 + jnp.exp(-a)))
        shb = jnp.dot(sh_ref[...], shb_ref[...], preferred_element_type=jnp.float32)
        y0 = shb[:, 0:16]
        yv0 = shb[:, 16:32]
        yv1 = shb[:, 32:48]
        yv2 = shb[:, 48:64]
        xs = xb[:, 0:16]
        xv0 = xb[:, 16:32]
        xv1 = xb[:, 32:48]
        xv2 = xb[:, 48:64]
        zA = xs * y0
        dot = xv0 * yv0 + xv1 * yv1 + xv2 * yv2
        g = jnp.concatenate(
            [zA, dot, xs, xv0, xv1, xv2, jnp.zeros((_EB, 32), jnp.float32)], axis=1)
        Q = jnp.dot(g.astype(jnp.bfloat16), wc_ref[...],
                    preferred_element_type=jnp.float32)
        H = jnp.dot(h.astype(jnp.bfloat16), rep_ref[...],
                    preferred_element_type=jnp.float32)
        S = Q[:, 16 * 128:17 * 128]
        for k in range(16):
            S = S + H[:, 128 * k:128 * (k + 1)] * Q[:, 128 * k:128 * (k + 1)]
        out0 = _INV * S[:, 0:16]
        t = S[:, 16:32]
        o1 = [
            _INV * (t * shb[:, 16 + 16 * k:32 + 16 * k] + y0 * S[:, 32 + 16 * k:48 + 16 * k])
            for k in range(3)
        ]
        o_ref[...] = jnp.concatenate(
            [out0] + o1
            + [jnp.ones((_EB, 16), jnp.float32), jnp.zeros((_EB, 48), jnp.float32)],
            axis=1)

    return pl.pallas_call(
        body,
        grid=(_E // _EB,),
        in_specs=[
            pl.BlockSpec((_EB, _W), lambda i: (i, 0)),
            pl.BlockSpec((_EB, 16), lambda i: (i, 0)),
            pl.BlockSpec((_EB, 4), lambda i: (i, 0)),
            pl.BlockSpec((16, 16), lambda i: (0, 0)),
            pl.BlockSpec((1, 16), lambda i: (0, 0)),
            pl.BlockSpec((128, 17 * 128), lambda i: (0, 0)),
            pl.BlockSpec((16, 16 * 128), lambda i: (0, 0)),
            pl.BlockSpec((4, 64), lambda i: (0, 0)),
        ],
        out_specs=pl.BlockSpec((_EB, _W), lambda i: (i, 0)),
        out_shape=jax.ShapeDtypeStruct((_E, _W), jnp.float32),
    )(x, ea, sh, W1, b1, Wc, Rep, Shb)


def _sc_scatter(y, dst2p, zrow):
    mesh = plsc.VectorSubcoreMesh(core_axis_name="c", subcore_axis_name="s")

    @functools.partial(
        pl.kernel,
        out_type=jax.ShapeDtypeStruct((_NC, _N, _W), jnp.float32),
        mesh=mesh,
        scratch_types=[
            pltpu.VMEM((8, _GC), jnp.int32),
            pltpu.VMEM((256, _W), jnp.float32),
            pltpu.VMEM_SHARED((_N, _W), jnp.float32),
            pltpu.SemaphoreType.DMA,
        ],
    )
    def sk(y_hbm, dst2_hbm, z_hbm, psum_hbm, idx_v, y_v, acc_sh, sem):
        c = lax.axis_index("c")
        s = lax.axis_index("s")
        wid = s * _NC + c

        @pl.when(s == 0)
        def _():
            pltpu.sync_copy(z_hbm, acc_sh)

        plsc.subcore_barrier()

        def group(gid):
            base = pl.multiple_of(gid * _G8, _G8)
            grow = pl.multiple_of(gid * 8, 8)
            pltpu.sync_copy(dst2_hbm.at[pl.ds(grow, 8)], idx_v)
            for q in range(4):
                pltpu.sync_copy(y_hbm.at[pl.ds(base + q * 256, 256)], y_v)
                descs = [pltpu.async_copy(
                    y_v.at[pl.ds(_GC * i, _GC)],
                    acc_sh.at[idx_v.at[q * 2 + i]], sem, add=True)
                    for i in range(2)]
                for d in descs:
                    d.wait()

        def body(j, carry):
            group(j * _NW + wid)
            return carry

        lax.fori_loop(0, _JF, body, 0)

        @pl.when(wid < _GEXT)
        def _():
            group(_JF * _NW + wid)

        @pl.when(wid == _GEXT)
        def _():
            pltpu.sync_copy(dst2_hbm.at[pl.ds(_NGF * 8, 8)], idx_v)
            pltpu.sync_copy(y_hbm.at[pl.ds(_REMB, _REMC)], y_v.at[pl.ds(0, _REMC)])
            descs = [pltpu.async_copy(
                y_v.at[pl.ds(_GC * i, _GC)],
                acc_sh.at[idx_v.at[i]], sem, add=True)
                for i in range(_REMC // _GC)]
            for d in descs:
                d.wait()

        plsc.subcore_barrier()

        nfull = _N // _GC
        rem = _N - nfull * _GC

        def dump(r0, nrows):
            pltpu.sync_copy(acc_sh.at[pl.ds(r0, nrows)], y_v.at[pl.ds(0, nrows)])
            pltpu.sync_copy(y_v.at[pl.ds(0, nrows)], psum_hbm.at[c, pl.ds(r0, nrows)])

        def dbody(j, carry):
            g = j * _NS + s

            @pl.when(g < nfull)
            def _():
                dump(pl.multiple_of(g * _GC, _GC), _GC)

            @pl.when(g == nfull)
            def _():
                dump(nfull * _GC, rem)

            return carry

        lax.fori_loop(0, (nfull + _NS) // _NS, dbody, 0)

    return sk(y, dst2p, zrow)


def _tc_finalize(psum):
    def body(ps_ref, o_ref):
        ssum = ps_ref[0, :, 0:64] + ps_ref[1, :, 0:64]
        cnt = ps_ref[0, :, 64:65] + ps_ref[1, :, 64:65]
        o_ref[...] = ssum / jnp.maximum(cnt, 1.0)

    return pl.pallas_call(
        body,
        out_shape=jax.ShapeDtypeStruct((_N, 64), jnp.float32),
    )(psum)


def kernel(node_attr, edge_index, edge_attr, edge_sh, W1, b1, W2, b2):
    na = jnp.pad(node_attr[:, _PERM], ((0, 0), (0, _W - 64)))
    src = edge_index[0]
    dst = edge_index[1]
    Wc = _assemble_wc(W2, b2)
    x = _sc_gather(na, src)
    y = _tc_dense(x, edge_attr, edge_sh, W1, b1.reshape(1, 16),
                  Wc.astype(jnp.bfloat16),
                  jnp.asarray(_REP, jnp.bfloat16), jnp.asarray(_SHB))
    zrow = jnp.zeros((_N, _W), jnp.float32)
    dst2p = jnp.pad(dst.reshape(_E // _GC, _GC), ((0, 6), (0, 0)))
    psum = _sc_scatter(y, dst2p, zrow)
    out = _tc_finalize(psum)
    return out[:, _COLMAP]


# speedup vs baseline: 1.0475x; 1.0475x over previous
import functools

import jax
import jax.numpy as jnp
import numpy as np
from jax import lax
from jax.experimental import pallas as pl
from jax.experimental.pallas import tpu as pltpu
from jax.experimental.pallas import tpu_sc as plsc

_N = 10000
_E = 160000

_NC, _NS = 2, 16
_NW = _NC * _NS
_GC = 128
_W = 128
_G8 = 1024
_NGF = _E // _G8
_JF = _NGF // _NW
_GEXT = _NGF - _JF * _NW
_REMB = _NGF * _G8
_REMC = _E - _REMB

_EB = 2000
_K = 17 * 128

_S3 = 1.0 / np.sqrt(3.0)
_INV = 1.0 / np.sqrt(32.0)

_PERM = np.concatenate([np.arange(16)] + [16 + 3 * np.arange(16) + k for k in range(3)])
_COLMAP = np.zeros(64, dtype=np.int32)
_COLMAP[:16] = np.arange(16)
for _w in range(16):
    for _k in range(3):
        _COLMAP[16 + 3 * _w + _k] = 16 + 16 * _k + _w


def _assemble_wc(W2, b2):
    W2r = W2.reshape(16, 4, 16, 16)
    b2r = b2.reshape(4, 16, 16)
    T = jnp.zeros((17, 128, 128), jnp.float32)
    T = T.at[:16, 0:16, 0:16].set(W2r[:, 0])
    T = T.at[:16, 16:32, 0:16].set(_S3 * W2r[:, 3])
    T = T.at[:16, 32:48, 16:32].set(W2r[:, 1])
    for kk in range(3):
        T = T.at[:16, 48 + 16 * kk:64 + 16 * kk, 32 + 16 * kk:48 + 16 * kk].set(W2r[:, 2])
    T = T.at[16, 0:16, 0:16].set(b2r[0])
    T = T.at[16, 16:32, 0:16].set(_S3 * b2r[3])
    T = T.at[16, 32:48, 16:32].set(b2r[1])
    for kk in range(3):
        T = T.at[16, 48 + 16 * kk:64 + 16 * kk, 32 + 16 * kk:48 + 16 * kk].set(b2r[2])
    return T.transpose(1, 0, 2).reshape(128, 17 * 128)


_REP = np.zeros((16, 16 * 128), np.float32)
for _kk in range(16):
    _REP[_kk, 128 * _kk:128 * (_kk + 1)] = 1.0
_SHB = np.zeros((4, 64), np.float32)
for _kk in range(4):
    _SHB[_kk, 16 * _kk:16 * (_kk + 1)] = 1.0


def _sc_gather(na, src):
    mesh = plsc.VectorSubcoreMesh(core_axis_name="c", subcore_axis_name="s")

    @functools.partial(
        pl.kernel,
        out_type=jax.ShapeDtypeStruct((_E, _W), jnp.float32),
        mesh=mesh,
        scratch_types=[
            pltpu.VMEM((_G8,), jnp.int32),
            pltpu.VMEM((512, _W), jnp.float32),
            pltpu.SemaphoreType.DMA,
        ],
    )
    def gk(na_hbm, src_hbm, x_hbm, idx_v, rows_v, sem):
        c = lax.axis_index("c")
        s = lax.axis_index("s")
        wid = s * _NC + c

        def group(gid):
            base = pl.multiple_of(gid * _G8, _G8)
            pltpu.sync_copy(src_hbm.at[pl.ds(base, _G8)], idx_v)
            for half in range(2):
                descs = [pltpu.async_copy(
                    na_hbm.at[idx_v.at[pl.ds(half * 512 + _GC * i, _GC)]],
                    rows_v.at[pl.ds(_GC * i, _GC)], sem) for i in range(4)]
                for d in descs:
                    d.wait()
                pltpu.sync_copy(rows_v, x_hbm.at[pl.ds(base + half * 512, 512)])

        def body(j, carry):
            group(j * _NW + wid)
            return carry

        lax.fori_loop(0, _JF, body, 0)

        @pl.when(wid < _GEXT)
        def _():
            group(_JF * _NW + wid)

        @pl.when(wid == _GEXT)
        def _():
            pltpu.sync_copy(src_hbm.at[pl.ds(_REMB, _REMC)], idx_v.at[pl.ds(0, _REMC)])
            descs = [pltpu.async_copy(
                na_hbm.at[idx_v.at[pl.ds(_GC * i, _GC)]],
                rows_v.at[pl.ds(_GC * i, _GC)], sem) for i in range(_REMC // _GC)]
            for d in descs:
                d.wait()
            pltpu.sync_copy(rows_v.at[pl.ds(0, _REMC)], x_hbm.at[pl.ds(_REMB, _REMC)])

    return gk(na, src)


def _tc_dense(x, ea, sh, W1, b1, Wc, Rep, Shb):
    def body(x_ref, ea_ref, sh_ref, w1_ref, b1_ref, wc_ref, rep_ref,
             shb_ref, o_ref):
        xb = x_ref[...]
        a = jnp.dot(ea_ref[...], w1_ref[...],
                    preferred_element_type=jnp.float32) + b1_ref[...]
        h = a * (1.0 / (1.0 + jnp.exp(-a)))
        shb = jnp.dot(sh_ref[...], shb_ref[...], preferred_element_type=jnp.float32)
        y0 = shb[:, 0:16]
        yv0 = shb[:, 16:32]
        yv1 = shb[:, 32:48]
        yv2 = shb[:, 48:64]
        xs = xb[:, 0:16]
        xv0 = xb[:, 16:32]
        xv1 = xb[:, 32:48]
        xv2 = xb[:, 48:64]
        zA = xs * y0
        dot = xv0 * yv0 + xv1 * yv1 + xv2 * yv2
        g = jnp.concatenate(
            [zA, dot, xs, xv0, xv1, xv2, jnp.zeros((_EB, 32), jnp.float32)], axis=1)
        Q = jnp.dot(g.astype(jnp.bfloat16), wc_ref[...],
                    preferred_element_type=jnp.float32)
        H = jnp.dot(h.astype(jnp.bfloat16), rep_ref[...],
                    preferred_element_type=jnp.float32)
        S = Q[:, 16 * 128:17 * 128]
        for k in range(16):
            S = S + H[:, 128 * k:128 * (k + 1)] * Q[:, 128 * k:128 * (k + 1)]
        out0 = _INV * S[:, 0:16]
        t = S[:, 16:32]
        o1 = [
            _INV * (t * shb[:, 16 + 16 * k:32 + 16 * k] + y0 * S[:, 32 + 16 * k:48 + 16 * k])
            for k in range(3)
        ]
        o_ref[...] = jnp.concatenate(
            [out0] + o1
            + [jnp.ones((_EB, 16), jnp.float32), jnp.zeros((_EB, 48), jnp.float32)],
            axis=1)

    return pl.pallas_call(
        body,
        grid=(_E // _EB,),
        in_specs=[
            pl.BlockSpec((_EB, _W), lambda i: (i, 0)),
            pl.BlockSpec((_EB, 16), lambda i: (i, 0)),
            pl.BlockSpec((_EB, 4), lambda i: (i, 0)),
            pl.BlockSpec((16, 16), lambda i: (0, 0)),
            pl.BlockSpec((1, 16), lambda i: (0, 0)),
            pl.BlockSpec((128, 17 * 128), lambda i: (0, 0)),
            pl.BlockSpec((16, 16 * 128), lambda i: (0, 0)),
            pl.BlockSpec((4, 64), lambda i: (0, 0)),
        ],
        out_specs=pl.BlockSpec((_EB, _W), lambda i: (i, 0)),
        out_shape=jax.ShapeDtypeStruct((_E, _W), jnp.float32),
    )(x, ea, sh, W1, b1, Wc, Rep, Shb)


def _sc_scatter(y, dst2p, zrow):
    mesh = plsc.VectorSubcoreMesh(core_axis_name="c", subcore_axis_name="s")

    @functools.partial(
        pl.kernel,
        out_type=jax.ShapeDtypeStruct((_NC, _N, _W), jnp.float32),
        mesh=mesh,
        scratch_types=[
            pltpu.VMEM((8, _GC), jnp.int32),
            pltpu.VMEM((256, _W), jnp.float32),
            pltpu.VMEM_SHARED((_N, _W), jnp.float32),
            pltpu.SemaphoreType.DMA,
        ],
    )
    def sk(y_hbm, dst2_hbm, z_hbm, psum_hbm, idx_v, y_v, acc_sh, sem):
        c = lax.axis_index("c")
        s = lax.axis_index("s")
        wid = s * _NC + c

        @pl.when(s == 0)
        def _():
            pltpu.sync_copy(z_hbm, acc_sh)

        plsc.subcore_barrier()

        def group(gid):
            base = pl.multiple_of(gid * _G8, _G8)
            grow = pl.multiple_of(gid * 8, 8)
            pltpu.sync_copy(dst2_hbm.at[pl.ds(grow, 8)], idx_v)
            for q in range(4):
                pltpu.sync_copy(y_hbm.at[pl.ds(base + q * 256, 256)], y_v)
                descs = [pltpu.async_copy(
                    y_v.at[pl.ds(_GC * i, _GC)],
                    acc_sh.at[idx_v.at[q * 2 + i]], sem, add=True)
                    for i in range(2)]
                for d in descs:
                    d.wait()

        def body(j, carry):
            group(j * _NW + wid)
            return carry

        lax.fori_loop(0, _JF, body, 0)

        @pl.when(wid < _GEXT)
        def _():
            group(_JF * _NW + wid)

        @pl.when(wid == _GEXT)
        def _():
            pltpu.sync_copy(dst2_hbm.at[pl.ds(_NGF * 8, 8)], idx_v)
            pltpu.sync_copy(y_hbm.at[pl.ds(_REMB, _REMC)], y_v.at[pl.ds(0, _REMC)])
            descs = [pltpu.async_copy(
                y_v.at[pl.ds(_GC * i, _GC)],
                acc_sh.at[idx_v.at[i]], sem, add=True)
                for i in range(_REMC // _GC)]
            for d in descs:
                d.wait()

        plsc.subcore_barrier()

        nfull = _N // _GC
        rem = _N - nfull * _GC

        def dump(r0, nrows):
            pltpu.sync_copy(acc_sh.at[pl.ds(r0, nrows)], y_v.at[pl.ds(0, nrows)])
            pltpu.sync_copy(y_v.at[pl.ds(0, nrows)], psum_hbm.at[c, pl.ds(r0, nrows)])

        def dbody(j, carry):
            g = j * _NS + s

            @pl.when(g < nfull)
            def _():
                dump(pl.multiple_of(g * _GC, _GC), _GC)

            @pl.when(g == nfull)
            def _():
                dump(nfull * _GC, rem)

            return carry

        lax.fori_loop(0, (nfull + _NS) // _NS, dbody, 0)

    return sk(y, dst2p, zrow)


def _tc_finalize(psum):
    def body(ps_ref, o_ref):
        ssum = ps_ref[0, :, 0:64] + ps_ref[1, :, 0:64]
        cnt = ps_ref[0, :, 64:65] + ps_ref[1, :, 64:65]
        o_ref[...] = ssum / jnp.maximum(cnt, 1.0)

    return pl.pallas_call(
        body,
        out_shape=jax.ShapeDtypeStruct((_N, 64), jnp.float32),
    )(psum)


def kernel(node_attr, edge_index, edge_attr, edge_sh, W1, b1, W2, b2):
    na = jnp.pad(node_attr[:, _PERM], ((0, 0), (0, _W - 64)))
    src = edge_index[0]
    dst = edge_index[1]
    Wc = _assemble_wc(W2, b2)
    x = _sc_gather(na, src)
    y = _tc_dense(x, edge_attr, edge_sh, W1, b1.reshape(1, 16),
                  Wc.astype(jnp.bfloat16),
                  jnp.asarray(_REP, jnp.bfloat16), jnp.asarray(_SHB))
    zrow = jnp.zeros((_N, _W), jnp.float32)
    dst2p = jnp.pad(dst.reshape(_E // _GC, _GC), ((0, 6), (0, 0)))
    psum = _sc_scatter(y, dst2p, zrow)
    out = _tc_finalize(psum)
    return out[:, _COLMAP]


# speedup vs baseline: 1.0577x; 1.0097x over previous
import functools

import jax
import jax.numpy as jnp
import numpy as np
from jax import lax
from jax.experimental import pallas as pl
from jax.experimental.pallas import tpu as pltpu
from jax.experimental.pallas import tpu_sc as plsc

_N = 10000
_E = 160000

_NC, _NS = 2, 16
_NW = _NC * _NS
_GC = 128
_W = 128
_G8 = 1024
_EH = _E // 2
_NGF = _EH // _G8
_JF = _NGF // _NW
_GEXT = _NGF - _JF * _NW
_REMB = _NGF * _G8
_REMC = _EH - _REMB

_EB = 1600
_K = 17 * 128

_S3 = 1.0 / np.sqrt(3.0)
_INV = 1.0 / np.sqrt(32.0)

_PERM = np.concatenate([np.arange(16)] + [16 + 3 * np.arange(16) + k for k in range(3)])
_COLMAP = np.zeros(64, dtype=np.int32)
_COLMAP[:16] = np.arange(16)
for _w in range(16):
    for _k in range(3):
        _COLMAP[16 + 3 * _w + _k] = 16 + 16 * _k + _w


def _assemble_wc(W2, b2):
    W2r = W2.reshape(16, 4, 16, 16)
    b2r = b2.reshape(4, 16, 16)
    T = jnp.zeros((17, 128, 128), jnp.float32)
    T = T.at[:16, 0:16, 0:16].set(W2r[:, 0])
    T = T.at[:16, 16:32, 0:16].set(_S3 * W2r[:, 3])
    T = T.at[:16, 32:48, 16:32].set(W2r[:, 1])
    for kk in range(3):
        T = T.at[:16, 48 + 16 * kk:64 + 16 * kk, 32 + 16 * kk:48 + 16 * kk].set(W2r[:, 2])
    T = T.at[16, 0:16, 0:16].set(b2r[0])
    T = T.at[16, 16:32, 0:16].set(_S3 * b2r[3])
    T = T.at[16, 32:48, 16:32].set(b2r[1])
    for kk in range(3):
        T = T.at[16, 48 + 16 * kk:64 + 16 * kk, 32 + 16 * kk:48 + 16 * kk].set(b2r[2])
    return T.transpose(1, 0, 2).reshape(128, 17 * 128)


_REP = np.zeros((16, 16 * 128), np.float32)
for _kk in range(16):
    _REP[_kk, 128 * _kk:128 * (_kk + 1)] = 1.0
_SHB = np.zeros((4, 64), np.float32)
for _kk in range(4):
    _SHB[_kk, 16 * _kk:16 * (_kk + 1)] = 1.0


def _sc_gather(na, src):
    mesh = plsc.VectorSubcoreMesh(core_axis_name="c", subcore_axis_name="s")

    @functools.partial(
        pl.kernel,
        out_type=jax.ShapeDtypeStruct((_EH, _W), jnp.float32),
        mesh=mesh,
        scratch_types=[
            pltpu.VMEM((_G8,), jnp.int32),
            pltpu.VMEM((512, _W), jnp.float32),
            pltpu.SemaphoreType.DMA,
        ],
    )
    def gk(na_hbm, src_hbm, x_hbm, idx_v, rows_v, sem):
        c = lax.axis_index("c")
        s = lax.axis_index("s")
        wid = s * _NC + c

        def group(gid):
            base = pl.multiple_of(gid * _G8, _G8)
            pltpu.sync_copy(src_hbm.at[pl.ds(base, _G8)], idx_v)
            for half in range(2):
                descs = [pltpu.async_copy(
                    na_hbm.at[idx_v.at[pl.ds(half * 512 + _GC * i, _GC)]],
                    rows_v.at[pl.ds(_GC * i, _GC)], sem) for i in range(4)]
                for d in descs:
                    d.wait()
                pltpu.sync_copy(rows_v, x_hbm.at[pl.ds(base + half * 512, 512)])

        def body(j, carry):
            group(j * _NW + wid)
            return carry

        lax.fori_loop(0, _JF, body, 0)

        @pl.when(wid < _GEXT)
        def _():
            group(_JF * _NW + wid)

        @pl.when(wid == _GEXT)
        def _():
            pltpu.sync_copy(src_hbm.at[pl.ds(_REMB, _REMC)], idx_v.at[pl.ds(0, _REMC)])
            descs = [pltpu.async_copy(
                na_hbm.at[idx_v.at[pl.ds(_GC * i, _GC)]],
                rows_v.at[pl.ds(_GC * i, _GC)], sem) for i in range(_REMC // _GC)]
            for d in descs:
                d.wait()
            pltpu.sync_copy(rows_v.at[pl.ds(0, _REMC)], x_hbm.at[pl.ds(_REMB, _REMC)])

    return gk(na, src)


def _tc_dense(x, ea, sh, W1, b1, Wc, Rep, Shb):
    def body(x_ref, ea_ref, sh_ref, w1_ref, b1_ref, wc_ref, rep_ref,
             shb_ref, o_ref):
        xb = x_ref[...]
        a = jnp.dot(ea_ref[...], w1_ref[...],
                    preferred_element_type=jnp.float32) + b1_ref[...]
        h = a * (1.0 / (1.0 + jnp.exp(-a)))
        shb = jnp.dot(sh_ref[...], shb_ref[...], preferred_element_type=jnp.float32)
        y0 = shb[:, 0:16]
        yv0 = shb[:, 16:32]
        yv1 = shb[:, 32:48]
        yv2 = shb[:, 48:64]
        xs = xb[:, 0:16]
        xv0 = xb[:, 16:32]
        xv1 = xb[:, 32:48]
        xv2 = xb[:, 48:64]
        zA = xs * y0
        dot = xv0 * yv0 + xv1 * yv1 + xv2 * yv2
        g = jnp.concatenate(
            [zA, dot, xs, xv0, xv1, xv2, jnp.zeros((_EB, 32), jnp.float32)], axis=1)
        Q = jnp.dot(g.astype(jnp.bfloat16), wc_ref[...],
                    preferred_element_type=jnp.float32)
        H = jnp.dot(h.astype(jnp.bfloat16), rep_ref[...],
                    preferred_element_type=jnp.float32)
        S = Q[:, 16 * 128:17 * 128]
        for k in range(16):
            S = S + H[:, 128 * k:128 * (k + 1)] * Q[:, 128 * k:128 * (k + 1)]
        out0 = _INV * S[:, 0:16]
        t = S[:, 16:32]
        o1 = [
            _INV * (t * shb[:, 16 + 16 * k:32 + 16 * k] + y0 * S[:, 32 + 16 * k:48 + 16 * k])
            for k in range(3)
        ]
        o_ref[...] = jnp.concatenate(
            [out0] + o1
            + [jnp.ones((_EB, 16), jnp.float32), jnp.zeros((_EB, 48), jnp.float32)],
            axis=1)

    return pl.pallas_call(
        body,
        grid=(_EH // _EB,),
        in_specs=[
            pl.BlockSpec((_EB, _W), lambda i: (i, 0)),
            pl.BlockSpec((_EB, 16), lambda i: (i, 0)),
            pl.BlockSpec((_EB, 4), lambda i: (i, 0)),
            pl.BlockSpec((16, 16), lambda i: (0, 0)),
            pl.BlockSpec((1, 16), lambda i: (0, 0)),
            pl.BlockSpec((128, 17 * 128), lambda i: (0, 0)),
            pl.BlockSpec((16, 16 * 128), lambda i: (0, 0)),
            pl.BlockSpec((4, 64), lambda i: (0, 0)),
        ],
        out_specs=pl.BlockSpec((_EB, _W), lambda i: (i, 0)),
        out_shape=jax.ShapeDtypeStruct((_EH, _W), jnp.float32),
    )(x, ea, sh, W1, b1, Wc, Rep, Shb)


def _sc_scatter(y, dst2p, zrow):
    mesh = plsc.VectorSubcoreMesh(core_axis_name="c", subcore_axis_name="s")

    @functools.partial(
        pl.kernel,
        out_type=jax.ShapeDtypeStruct((_NC, _N, _W), jnp.float32),
        mesh=mesh,
        scratch_types=[
            pltpu.VMEM((8, _GC), jnp.int32),
            pltpu.VMEM((256, _W), jnp.float32),
            pltpu.VMEM_SHARED((_N, _W), jnp.float32),
            pltpu.SemaphoreType.DMA,
        ],
    )
    def sk(y_hbm, dst2_hbm, z_hbm, psum_hbm, idx_v, y_v, acc_sh, sem):
        c = lax.axis_index("c")
        s = lax.axis_index("s")
        wid = s * _NC + c

        @pl.when(s == 0)
        def _():
            pltpu.sync_copy(z_hbm, acc_sh)

        plsc.subcore_barrier()

        def group(gid):
            base = pl.multiple_of(gid * _G8, _G8)
            grow = pl.multiple_of(gid * 8, 8)
            pltpu.sync_copy(dst2_hbm.at[pl.ds(grow, 8)], idx_v)
            for q in range(4):
                pltpu.sync_copy(y_hbm.at[pl.ds(base + q * 256, 256)], y_v)
                descs = [pltpu.async_copy(
                    y_v.at[pl.ds(_GC * i, _GC)],
                    acc_sh.at[idx_v.at[q * 2 + i]], sem, add=True)
                    for i in range(2)]
                for d in descs:
                    d.wait()

        def body(j, carry):
            group(j * _NW + wid)
            return carry

        lax.fori_loop(0, _JF, body, 0)

        @pl.when(wid < _GEXT)
        def _():
            group(_JF * _NW + wid)

        @pl.when(wid == _GEXT)
        def _():
            pltpu.sync_copy(dst2_hbm.at[pl.ds(_NGF * 8, 8)], idx_v)
            pltpu.sync_copy(y_hbm.at[pl.ds(_REMB, _REMC)], y_v.at[pl.ds(0, _REMC)])
            descs = [pltpu.async_copy(
                y_v.at[pl.ds(_GC * i, _GC)],
                acc_sh.at[idx_v.at[i]], sem, add=True)
                for i in range(_REMC // _GC)]
            for d in descs:
                d.wait()

        plsc.subcore_barrier()

        nfull = _N // _GC
        rem = _N - nfull * _GC

        def dump(r0, nrows):
            pltpu.sync_copy(acc_sh.at[pl.ds(r0, nrows)], y_v.at[pl.ds(0, nrows)])
            pltpu.sync_copy(y_v.at[pl.ds(0, nrows)], psum_hbm.at[c, pl.ds(r0, nrows)])

        def dbody(j, carry):
            g = j * _NS + s

            @pl.when(g < nfull)
            def _():
                dump(pl.multiple_of(g * _GC, _GC), _GC)

            @pl.when(g == nfull)
            def _():
                dump(nfull * _GC, rem)

            return carry

        lax.fori_loop(0, (nfull + _NS) // _NS, dbody, 0)

    return sk(y, dst2p, zrow)


def _tc_finalize(psum1, psum2):
    def body(p1_ref, p2_ref, o_ref):
        ssum = (p1_ref[0, :, 0:64] + p1_ref[1, :, 0:64]
                + p2_ref[0, :, 0:64] + p2_ref[1, :, 0:64])
        cnt = (p1_ref[0, :, 64:65] + p1_ref[1, :, 64:65]
               + p2_ref[0, :, 64:65] + p2_ref[1, :, 64:65])
        o_ref[...] = ssum / jnp.maximum(cnt, 1.0)

    return pl.pallas_call(
        body,
        out_shape=jax.ShapeDtypeStruct((_N, 64), jnp.float32),
    )(psum1, psum2)


def kernel(node_attr, edge_index, edge_attr, edge_sh, W1, b1, W2, b2):
    na = jnp.pad(node_attr[:, _PERM], ((0, 0), (0, _W - 64)))
    src = edge_index[0]
    dst = edge_index[1]
    Wc = _assemble_wc(W2, b2).astype(jnp.bfloat16)
    Rep = jnp.asarray(_REP, jnp.bfloat16)
    Shb = jnp.asarray(_SHB)
    b1r = b1.reshape(1, 16)
    zrow = jnp.zeros((_N, _W), jnp.float32)

    def dst2pad(d1):
        return jnp.pad(d1.reshape(_EH // _GC, _GC), ((0, 7), (0, 0)))

    x1 = _sc_gather(na, src[:_EH])
    x2 = _sc_gather(na, src[_EH:])
    y1 = _tc_dense(x1, edge_attr[:_EH], edge_sh[:_EH], W1, b1r, Wc, Rep, Shb)
    psum1 = _sc_scatter(y1, dst2pad(dst[:_EH]), zrow)
    y2 = _tc_dense(x2, edge_attr[_EH:], edge_sh[_EH:], W1, b1r, Wc, Rep, Shb)
    psum2 = _sc_scatter(y2, dst2pad(dst[_EH:]), zrow)
    out = _tc_finalize(psum1, psum2)
    return out[:, _COLMAP]


# speedup vs baseline: 1.1014x; 1.0413x over previous
import functools

import jax
import jax.numpy as jnp
import numpy as np
from jax import lax
from jax.experimental import pallas as pl
from jax.experimental.pallas import tpu as pltpu
from jax.experimental.pallas import tpu_sc as plsc

_N = 10000
_E = 160000

_NC, _NS = 2, 16
_NW = _NC * _NS
_GC = 128
_W = 128
_G8 = 1024
_EH = _E // 2
_NGF = _EH // _G8
_JF = _NGF // _NW
_GEXT = _NGF - _JF * _NW
_REMB = _NGF * _G8
_REMC = _EH - _REMB

_EB = 1600
_K = 17 * 128

_S3 = 1.0 / np.sqrt(3.0)
_INV = 1.0 / np.sqrt(32.0)

_PERM = np.concatenate([np.arange(16)] + [16 + 3 * np.arange(16) + k for k in range(3)])
_COLMAP = np.zeros(64, dtype=np.int32)
_COLMAP[:16] = np.arange(16)
for _w in range(16):
    for _k in range(3):
        _COLMAP[16 + 3 * _w + _k] = 16 + 16 * _k + _w


def _assemble_wc(W2, b2):
    W2r = W2.reshape(16, 4, 16, 16)
    b2r = b2.reshape(4, 16, 16)
    T = jnp.zeros((17, 128, 128), jnp.float32)
    T = T.at[:16, 0:16, 0:16].set(W2r[:, 0])
    T = T.at[:16, 16:32, 0:16].set(_S3 * W2r[:, 3])
    T = T.at[:16, 32:48, 16:32].set(W2r[:, 1])
    for kk in range(3):
        T = T.at[:16, 48 + 16 * kk:64 + 16 * kk, 32 + 16 * kk:48 + 16 * kk].set(W2r[:, 2])
    T = T.at[16, 0:16, 0:16].set(b2r[0])
    T = T.at[16, 16:32, 0:16].set(_S3 * b2r[3])
    T = T.at[16, 32:48, 16:32].set(b2r[1])
    for kk in range(3):
        T = T.at[16, 48 + 16 * kk:64 + 16 * kk, 32 + 16 * kk:48 + 16 * kk].set(b2r[2])
    return T.transpose(1, 0, 2).reshape(128, 17 * 128)


_REP = np.zeros((16, 16 * 128), np.float32)
for _kk in range(16):
    _REP[_kk, 128 * _kk:128 * (_kk + 1)] = 1.0
_SHB = np.zeros((4, 64), np.float32)
for _kk in range(4):
    _SHB[_kk, 16 * _kk:16 * (_kk + 1)] = 1.0


def _sc_gather(na, src):
    mesh = plsc.VectorSubcoreMesh(core_axis_name="c", subcore_axis_name="s")

    @functools.partial(
        pl.kernel,
        out_type=jax.ShapeDtypeStruct((_EH, _W), jnp.float32),
        mesh=mesh,
        scratch_types=[
            pltpu.VMEM((_G8,), jnp.int32),
            pltpu.VMEM((512, _W), jnp.float32),
            pltpu.SemaphoreType.DMA,
        ],
    )
    def gk(na_hbm, src_hbm, x_hbm, idx_v, rows_v, sem):
        c = lax.axis_index("c")
        s = lax.axis_index("s")
        wid = s * _NC + c

        def group(gid):
            base = pl.multiple_of(gid * _G8, _G8)
            pltpu.sync_copy(src_hbm.at[pl.ds(base, _G8)], idx_v)
            for half in range(2):
                descs = [pltpu.async_copy(
                    na_hbm.at[idx_v.at[pl.ds(half * 512 + _GC * i, _GC)]],
                    rows_v.at[pl.ds(_GC * i, _GC)], sem) for i in range(4)]
                for d in descs:
                    d.wait()
                pltpu.sync_copy(rows_v, x_hbm.at[pl.ds(base + half * 512, 512)])

        def body(j, carry):
            group(j * _NW + wid)
            return carry

        lax.fori_loop(0, _JF, body, 0)

        @pl.when(wid < _GEXT)
        def _():
            group(_JF * _NW + wid)

        @pl.when(wid == _GEXT)
        def _():
            pltpu.sync_copy(src_hbm.at[pl.ds(_REMB, _REMC)], idx_v.at[pl.ds(0, _REMC)])
            descs = [pltpu.async_copy(
                na_hbm.at[idx_v.at[pl.ds(_GC * i, _GC)]],
                rows_v.at[pl.ds(_GC * i, _GC)], sem) for i in range(_REMC // _GC)]
            for d in descs:
                d.wait()
            pltpu.sync_copy(rows_v.at[pl.ds(0, _REMC)], x_hbm.at[pl.ds(_REMB, _REMC)])

    return gk(na, src)


def _tc_dense(x, easb, W1, b1, Wc, Rep, Shb):
    def body(x_ref, easb_ref, w1_ref, b1_ref, wc_ref, rep_ref,
             shb_ref, o_ref):
        xb = x_ref[...]
        easbb = easb_ref[...]
        a = jnp.dot(easbb[:, 0:16], w1_ref[...],
                    preferred_element_type=jnp.float32) + b1_ref[...]
        h = a * (1.0 / (1.0 + jnp.exp(-a)))
        shb = jnp.dot(easbb[:, 16:20], shb_ref[...], preferred_element_type=jnp.float32)
        y0 = shb[:, 0:16]
        yv0 = shb[:, 16:32]
        yv1 = shb[:, 32:48]
        yv2 = shb[:, 48:64]
        xs = xb[:, 0:16]
        xv0 = xb[:, 16:32]
        xv1 = xb[:, 32:48]
        xv2 = xb[:, 48:64]
        zA = xs * y0
        dot = xv0 * yv0 + xv1 * yv1 + xv2 * yv2
        g = jnp.concatenate(
            [zA, dot, xs, xv0, xv1, xv2, jnp.zeros((_EB, 32), jnp.float32)], axis=1)
        Q = jnp.dot(g.astype(jnp.bfloat16), wc_ref[...],
                    preferred_element_type=jnp.float32)
        H = jnp.dot(h.astype(jnp.bfloat16), rep_ref[...],
                    preferred_element_type=jnp.float32)
        S = Q[:, 16 * 128:17 * 128]
        for k in range(16):
            S = S + H[:, 128 * k:128 * (k + 1)] * Q[:, 128 * k:128 * (k + 1)]
        out0 = _INV * S[:, 0:16]
        t = S[:, 16:32]
        o1 = [
            _INV * (t * shb[:, 16 + 16 * k:32 + 16 * k] + y0 * S[:, 32 + 16 * k:48 + 16 * k])
            for k in range(3)
        ]
        o_ref[...] = jnp.concatenate(
            [out0] + o1
            + [jnp.ones((_EB, 16), jnp.float32), jnp.zeros((_EB, 48), jnp.float32)],
            axis=1)

    return pl.pallas_call(
        body,
        grid=(_EH // _EB,),
        in_specs=[
            pl.BlockSpec((_EB, _W), lambda i: (i, 0)),
            pl.BlockSpec((_EB, 20), lambda i: (i, 0)),
            pl.BlockSpec((16, 16), lambda i: (0, 0)),
            pl.BlockSpec((1, 16), lambda i: (0, 0)),
            pl.BlockSpec((128, 17 * 128), lambda i: (0, 0)),
            pl.BlockSpec((16, 16 * 128), lambda i: (0, 0)),
            pl.BlockSpec((4, 64), lambda i: (0, 0)),
        ],
        out_specs=pl.BlockSpec((_EB, _W), lambda i: (i, 0)),
        out_shape=jax.ShapeDtypeStruct((_EH, _W), jnp.float32),
    )(x, easb, W1, b1, Wc, Rep, Shb)


def _sc_scatter(y, dst2p, zrow):
    mesh = plsc.VectorSubcoreMesh(core_axis_name="c", subcore_axis_name="s")

    @functools.partial(
        pl.kernel,
        out_type=jax.ShapeDtypeStruct((_NC, _N, _W), jnp.float32),
        mesh=mesh,
        scratch_types=[
            pltpu.VMEM((8, _GC), jnp.int32),
            pltpu.VMEM((256, _W), jnp.float32),
            pltpu.VMEM_SHARED((_N, _W), jnp.float32),
            pltpu.SemaphoreType.DMA,
        ],
    )
    def sk(y_hbm, dst2_hbm, z_hbm, psum_hbm, idx_v, y_v, acc_sh, sem):
        c = lax.axis_index("c")
        s = lax.axis_index("s")
        wid = s * _NC + c

        @pl.when(s == 0)
        def _():
            pltpu.sync_copy(z_hbm, acc_sh)

        plsc.subcore_barrier()

        def group(gid):
            base = pl.multiple_of(gid * _G8, _G8)
            grow = pl.multiple_of(gid * 8, 8)
            pltpu.sync_copy(dst2_hbm.at[pl.ds(grow, 8)], idx_v)
            for q in range(4):
                pltpu.sync_copy(y_hbm.at[pl.ds(base + q * 256, 256)], y_v)
                descs = [pltpu.async_copy(
                    y_v.at[pl.ds(_GC * i, _GC)],
                    acc_sh.at[idx_v.at[q * 2 + i]], sem, add=True)
                    for i in range(2)]
                for d in descs:
                    d.wait()

        def body(j, carry):
            group(j * _NW + wid)
            return carry

        lax.fori_loop(0, _JF, body, 0)

        @pl.when(wid < _GEXT)
        def _():
            group(_JF * _NW + wid)

        @pl.when(wid == _GEXT)
        def _():
            pltpu.sync_copy(dst2_hbm.at[pl.ds(_NGF * 8, 8)], idx_v)
            pltpu.sync_copy(y_hbm.at[pl.ds(_REMB, _REMC)], y_v.at[pl.ds(0, _REMC)])
            descs = [pltpu.async_copy(
                y_v.at[pl.ds(_GC * i, _GC)],
                acc_sh.at[idx_v.at[i]], sem, add=True)
                for i in range(_REMC // _GC)]
            for d in descs:
                d.wait()

        plsc.subcore_barrier()

        nfull = _N // _GC
        rem = _N - nfull * _GC

        def dump(r0, nrows):
            pltpu.sync_copy(acc_sh.at[pl.ds(r0, nrows)], y_v.at[pl.ds(0, nrows)])
            pltpu.sync_copy(y_v.at[pl.ds(0, nrows)], psum_hbm.at[c, pl.ds(r0, nrows)])

        def dbody(j, carry):
            g = j * _NS + s

            @pl.when(g < nfull)
            def _():
                dump(pl.multiple_of(g * _GC, _GC), _GC)

            @pl.when(g == nfull)
            def _():
                dump(nfull * _GC, rem)

            return carry

        lax.fori_loop(0, (nfull + _NS) // _NS, dbody, 0)

    return sk(y, dst2p, zrow)


def _tc_finalize(psum1, psum2):
    def body(p1_ref, p2_ref, o_ref):
        ssum = (p1_ref[0, :, 0:64] + p1_ref[1, :, 0:64]
                + p2_ref[0, :, 0:64] + p2_ref[1, :, 0:64])
        cnt = (p1_ref[0, :, 64:65] + p1_ref[1, :, 64:65]
               + p2_ref[0, :, 64:65] + p2_ref[1, :, 64:65])
        o_ref[...] = ssum / jnp.maximum(cnt, 1.0)

    return pl.pallas_call(
        body,
        out_shape=jax.ShapeDtypeStruct((_N, 64), jnp.float32),
    )(psum1, psum2)


def kernel(node_attr, edge_index, edge_attr, edge_sh, W1, b1, W2, b2):
    na = jnp.pad(node_attr[:, _PERM], ((0, 0), (0, _W - 64)))
    src = edge_index[0]
    dst = edge_index[1]
    Wc = _assemble_wc(W2, b2).astype(jnp.bfloat16)
    Rep = jnp.asarray(_REP, jnp.bfloat16)
    Shb = jnp.asarray(_SHB)
    b1r = b1.reshape(1, 16)
    zrow = jnp.zeros((_N, _W), jnp.float32)

    def dst2pad(d1):
        return jnp.pad(d1.reshape(_EH // _GC, _GC), ((0, 7), (0, 0)))

    easb = jnp.concatenate([edge_attr, edge_sh], axis=1)
    x1 = _sc_gather(na, src[:_EH])
    x2 = _sc_gather(na, src[_EH:])
    y1 = _tc_dense(x1, easb[:_EH], W1, b1r, Wc, Rep, Shb)
    psum1 = _sc_scatter(y1, dst2pad(dst[:_EH]), zrow)
    y2 = _tc_dense(x2, easb[_EH:], W1, b1r, Wc, Rep, Shb)
    psum2 = _sc_scatter(y2, dst2pad(dst[_EH:]), zrow)
    out = _tc_finalize(psum1, psum2)
    return out[:, _COLMAP]


# speedup vs baseline: 1.1058x; 1.0040x over previous
import functools

import jax
import jax.numpy as jnp
import numpy as np
from jax import lax
from jax.experimental import pallas as pl
from jax.experimental.pallas import tpu as pltpu
from jax.experimental.pallas import tpu_sc as plsc

_N = 10000
_E = 160000

_NC, _NS = 2, 16
_NW = _NC * _NS
_GC = 128
_W = 128
_G8 = 1024
_EH = _E // 2
_NGF = _EH // _G8
_JF = _NGF // _NW
_GEXT = _NGF - _JF * _NW
_REMB = _NGF * _G8
_REMC = _EH - _REMB

_EB = 1600
_K = 17 * 128

_S3 = 1.0 / np.sqrt(3.0)
_INV = 1.0 / np.sqrt(32.0)

_PERM = np.concatenate([np.arange(16)] + [16 + 3 * np.arange(16) + k for k in range(3)])
_COLMAP = np.zeros(64, dtype=np.int32)
_COLMAP[:16] = np.arange(16)
for _w in range(16):
    for _k in range(3):
        _COLMAP[16 + 3 * _w + _k] = 16 + 16 * _k + _w


def _assemble_wc(W2, b2):
    W2r = W2.reshape(16, 4, 16, 16)
    b2r = b2.reshape(4, 16, 16)
    T = jnp.zeros((17, 128, 128), jnp.float32)
    T = T.at[:16, 0:16, 0:16].set(W2r[:, 0])
    T = T.at[:16, 16:32, 0:16].set(_S3 * W2r[:, 3])
    T = T.at[:16, 32:48, 16:32].set(W2r[:, 1])
    for kk in range(3):
        T = T.at[:16, 48 + 16 * kk:64 + 16 * kk, 32 + 16 * kk:48 + 16 * kk].set(W2r[:, 2])
    T = T.at[16, 0:16, 0:16].set(b2r[0])
    T = T.at[16, 16:32, 0:16].set(_S3 * b2r[3])
    T = T.at[16, 32:48, 16:32].set(b2r[1])
    for kk in range(3):
        T = T.at[16, 48 + 16 * kk:64 + 16 * kk, 32 + 16 * kk:48 + 16 * kk].set(b2r[2])
    return T.transpose(1, 0, 2).reshape(128, 17 * 128)


_REP = np.zeros((16, 16 * 128), np.float32)
for _kk in range(16):
    _REP[_kk, 128 * _kk:128 * (_kk + 1)] = 1.0
_SHB = np.zeros((4, 64), np.float32)
for _kk in range(4):
    _SHB[_kk, 16 * _kk:16 * (_kk + 1)] = 1.0


def _sc_gather(na, src):
    mesh = plsc.VectorSubcoreMesh(core_axis_name="c", subcore_axis_name="s")

    @functools.partial(
        pl.kernel,
        out_type=jax.ShapeDtypeStruct((_EH, _W), jnp.float32),
        mesh=mesh,
        scratch_types=[
            pltpu.VMEM((_G8,), jnp.int32),
            pltpu.VMEM((512, _W), jnp.float32),
            pltpu.SemaphoreType.DMA,
        ],
    )
    def gk(na_hbm, src_hbm, x_hbm, idx_v, rows_v, sem):
        c = lax.axis_index("c")
        s = lax.axis_index("s")
        wid = s * _NC + c

        def group(gid):
            base = pl.multiple_of(gid * _G8, _G8)
            pltpu.sync_copy(src_hbm.at[pl.ds(base, _G8)], idx_v)
            for half in range(2):
                descs = [pltpu.async_copy(
                    na_hbm.at[idx_v.at[pl.ds(half * 512 + _GC * i, _GC)]],
                    rows_v.at[pl.ds(_GC * i, _GC)], sem) for i in range(4)]
                for d in descs:
                    d.wait()
                pltpu.sync_copy(rows_v, x_hbm.at[pl.ds(base + half * 512, 512)])

        def body(j, carry):
            group(j * _NW + wid)
            return carry

        lax.fori_loop(0, _JF, body, 0)

        @pl.when(wid < _GEXT)
        def _():
            group(_JF * _NW + wid)

        @pl.when(wid == _GEXT)
        def _():
            pltpu.sync_copy(src_hbm.at[pl.ds(_REMB, _REMC)], idx_v.at[pl.ds(0, _REMC)])
            descs = [pltpu.async_copy(
                na_hbm.at[idx_v.at[pl.ds(_GC * i, _GC)]],
                rows_v.at[pl.ds(_GC * i, _GC)], sem) for i in range(_REMC // _GC)]
            for d in descs:
                d.wait()
            pltpu.sync_copy(rows_v.at[pl.ds(0, _REMC)], x_hbm.at[pl.ds(_REMB, _REMC)])

    return gk(na, src)


def _tc_dense(x, easb, W1, b1, Wc, Rep, Shb):
    def body(x_ref, easb_ref, w1_ref, b1_ref, wc_ref, rep_ref,
             shb_ref, o_ref):
        xb = x_ref[...]
        easbb = easb_ref[...]
        a = jnp.dot(easbb[:, 0:16], w1_ref[...],
                    preferred_element_type=jnp.float32) + b1_ref[...]
        h = a * (1.0 / (1.0 + jnp.exp(-a)))
        shb = jnp.dot(easbb[:, 16:20], shb_ref[...], preferred_element_type=jnp.float32)
        y0 = shb[:, 0:16]
        yv0 = shb[:, 16:32]
        yv1 = shb[:, 32:48]
        yv2 = shb[:, 48:64]
        xs = xb[:, 0:16]
        xv0 = xb[:, 16:32]
        xv1 = xb[:, 32:48]
        xv2 = xb[:, 48:64]
        zA = xs * y0
        dot = xv0 * yv0 + xv1 * yv1 + xv2 * yv2
        g = jnp.concatenate(
            [zA, dot, xs, xv0, xv1, xv2, jnp.zeros((_EB, 32), jnp.float32)], axis=1)
        Q = jnp.dot(g.astype(jnp.bfloat16), wc_ref[...],
                    preferred_element_type=jnp.float32)
        H = jnp.dot(h.astype(jnp.bfloat16), rep_ref[...],
                    preferred_element_type=jnp.float32)
        S = Q[:, 16 * 128:17 * 128]
        for k in range(16):
            S = S + H[:, 128 * k:128 * (k + 1)] * Q[:, 128 * k:128 * (k + 1)]
        out0 = _INV * S[:, 0:16]
        t = S[:, 16:32]
        o1 = [
            _INV * (t * shb[:, 16 + 16 * k:32 + 16 * k] + y0 * S[:, 32 + 16 * k:48 + 16 * k])
            for k in range(3)
        ]
        o_ref[...] = jnp.concatenate(
            [out0] + o1
            + [jnp.ones((_EB, 16), jnp.float32), jnp.zeros((_EB, 48), jnp.float32)],
            axis=1)

    return pl.pallas_call(
        body,
        grid=(_EH // _EB,),
        in_specs=[
            pl.BlockSpec((_EB, _W), lambda i: (i, 0)),
            pl.BlockSpec((_EB, 20), lambda i: (i, 0)),
            pl.BlockSpec((16, 16), lambda i: (0, 0)),
            pl.BlockSpec((1, 16), lambda i: (0, 0)),
            pl.BlockSpec((128, 17 * 128), lambda i: (0, 0)),
            pl.BlockSpec((16, 16 * 128), lambda i: (0, 0)),
            pl.BlockSpec((4, 64), lambda i: (0, 0)),
        ],
        out_specs=pl.BlockSpec((_EB, _W), lambda i: (i, 0)),
        out_shape=jax.ShapeDtypeStruct((_EH, _W), jnp.float32),
    )(x, easb, W1, b1, Wc, Rep, Shb)


def _sc_scatter(y, dst2p, zrow):
    mesh = plsc.VectorSubcoreMesh(core_axis_name="c", subcore_axis_name="s")

    @functools.partial(
        pl.kernel,
        out_type=jax.ShapeDtypeStruct((_NC, _N, _W), jnp.float32),
        mesh=mesh,
        scratch_types=[
            pltpu.VMEM((8, _GC), jnp.int32),
            pltpu.VMEM((256, _W), jnp.float32),
            pltpu.VMEM_SHARED((_N, _W), jnp.float32),
            pltpu.SemaphoreType.DMA,
        ],
    )
    def sk(y_hbm, dst2_hbm, z_hbm, psum_hbm, idx_v, y_v, acc_sh, sem):
        c = lax.axis_index("c")
        s = lax.axis_index("s")
        wid = s * _NC + c

        @pl.when(s == 0)
        def _():
            pltpu.sync_copy(z_hbm, acc_sh)

        plsc.subcore_barrier()

        def group(gid):
            base = pl.multiple_of(gid * _G8, _G8)
            grow = pl.multiple_of(gid * 8, 8)
            pltpu.sync_copy(dst2_hbm.at[pl.ds(grow, 8)], idx_v)
            for q in range(4):
                pltpu.sync_copy(y_hbm.at[pl.ds(base + q * 256, 256)], y_v)
                descs = [pltpu.async_copy(
                    y_v.at[pl.ds(_GC * i, _GC)],
                    acc_sh.at[idx_v.at[q * 2 + i]], sem, add=True)
                    for i in range(2)]
                for d in descs:
                    d.wait()

        def body(j, carry):
            group(j * _NW + wid)
            return carry

        lax.fori_loop(0, _JF, body, 0)

        @pl.when(wid < _GEXT)
        def _():
            group(_JF * _NW + wid)

        @pl.when(wid == _GEXT)
        def _():
            pltpu.sync_copy(dst2_hbm.at[pl.ds(_NGF * 8, 8)], idx_v)
            pltpu.sync_copy(y_hbm.at[pl.ds(_REMB, _REMC)], y_v.at[pl.ds(0, _REMC)])
            descs = [pltpu.async_copy(
                y_v.at[pl.ds(_GC * i, _GC)],
                acc_sh.at[idx_v.at[i]], sem, add=True)
                for i in range(_REMC // _GC)]
            for d in descs:
                d.wait()

        plsc.subcore_barrier()

        nfull = _N // _GC
        rem = _N - nfull * _GC

        def dump(r0, nrows):
            pltpu.sync_copy(acc_sh.at[pl.ds(r0, nrows)], y_v.at[pl.ds(0, nrows)])
            pltpu.sync_copy(y_v.at[pl.ds(0, nrows)], psum_hbm.at[c, pl.ds(r0, nrows)])

        def dbody(j, carry):
            g = j * _NS + s

            @pl.when(g < nfull)
            def _():
                dump(pl.multiple_of(g * _GC, _GC), _GC)

            @pl.when(g == nfull)
            def _():
                dump(nfull * _GC, rem)

            return carry

        lax.fori_loop(0, (nfull + _NS) // _NS, dbody, 0)

    return sk(y, dst2p, zrow)


_CPM = np.zeros((64, 64), np.float32)
for _j in range(64):
    _CPM[_COLMAP[_j], _j] = 1.0


def _tc_finalize(psum1, psum2):
    def body(p1_ref, p2_ref, pm_ref, o_ref):
        ssum = (p1_ref[0, :, 0:64] + p1_ref[1, :, 0:64]
                + p2_ref[0, :, 0:64] + p2_ref[1, :, 0:64])
        cnt = (p1_ref[0, :, 64:65] + p1_ref[1, :, 64:65]
               + p2_ref[0, :, 64:65] + p2_ref[1, :, 64:65])
        o_ref[...] = jnp.dot(ssum / jnp.maximum(cnt, 1.0), pm_ref[...],
                             preferred_element_type=jnp.float32)

    return pl.pallas_call(
        body,
        out_shape=jax.ShapeDtypeStruct((_N, 64), jnp.float32),
    )(psum1, psum2, jnp.asarray(_CPM))


def kernel(node_attr, edge_index, edge_attr, edge_sh, W1, b1, W2, b2):
    na = jnp.pad(node_attr[:, _PERM], ((0, 0), (0, _W - 64)))
    src = edge_index[0]
    dst = edge_index[1]
    Wc = _assemble_wc(W2, b2).astype(jnp.bfloat16)
    Rep = jnp.asarray(_REP, jnp.bfloat16)
    Shb = jnp.asarray(_SHB)
    b1r = b1.reshape(1, 16)
    zrow = jnp.zeros((_N, _W), jnp.float32)

    def dst2pad(d1):
        return jnp.pad(d1.reshape(_EH // _GC, _GC), ((0, 7), (0, 0)))

    easb = jnp.concatenate([edge_attr, edge_sh], axis=1)
    x1 = _sc_gather(na, src[:_EH])
    x2 = _sc_gather(na, src[_EH:])
    y1 = _tc_dense(x1, easb[:_EH], W1, b1r, Wc, Rep, Shb)
    psum1 = _sc_scatter(y1, dst2pad(dst[:_EH]), zrow)
    y2 = _tc_dense(x2, easb[_EH:], W1, b1r, Wc, Rep, Shb)
    psum2 = _sc_scatter(y2, dst2pad(dst[_EH:]), zrow)
    return _tc_finalize(psum1, psum2)


# speedup vs baseline: 1.1221x; 1.0147x over previous
import functools

import jax
import jax.numpy as jnp
import numpy as np
from jax import lax
from jax.experimental import pallas as pl
from jax.experimental.pallas import tpu as pltpu
from jax.experimental.pallas import tpu_sc as plsc

_N = 10000
_E = 160000

_NC, _NS = 2, 16
_NW = _NC * _NS
_GC = 128
_W = 128
_G8 = 1024
_EH = _E // 2
_NGF = _EH // _G8
_JF = _NGF // _NW
_GEXT = _NGF - _JF * _NW
_REMB = _NGF * _G8
_REMC = _EH - _REMB

_EB = 1600
_K = 17 * 128

_S3 = 1.0 / np.sqrt(3.0)
_INV = 1.0 / np.sqrt(32.0)

_PERM = np.concatenate([np.arange(16)] + [16 + 3 * np.arange(16) + k for k in range(3)])
_COLMAP = np.zeros(64, dtype=np.int32)
_COLMAP[:16] = np.arange(16)
for _w in range(16):
    for _k in range(3):
        _COLMAP[16 + 3 * _w + _k] = 16 + 16 * _k + _w


def _assemble_wc(W2, b2):
    W2r = W2.reshape(16, 4, 16, 16)
    b2r = b2.reshape(4, 16, 16)
    T = jnp.zeros((17, 128, 128), jnp.float32)
    T = T.at[:16, 0:16, 0:16].set(W2r[:, 0])
    T = T.at[:16, 16:32, 0:16].set(_S3 * W2r[:, 3])
    T = T.at[:16, 32:48, 16:32].set(W2r[:, 1])
    for kk in range(3):
        T = T.at[:16, 48 + 16 * kk:64 + 16 * kk, 32 + 16 * kk:48 + 16 * kk].set(W2r[:, 2])
    T = T.at[16, 0:16, 0:16].set(b2r[0])
    T = T.at[16, 16:32, 0:16].set(_S3 * b2r[3])
    T = T.at[16, 32:48, 16:32].set(b2r[1])
    for kk in range(3):
        T = T.at[16, 48 + 16 * kk:64 + 16 * kk, 32 + 16 * kk:48 + 16 * kk].set(b2r[2])
    return T.transpose(1, 0, 2).reshape(128, 17 * 128)


_REP = np.zeros((16, 16 * 128), np.float32)
for _kk in range(16):
    _REP[_kk, 128 * _kk:128 * (_kk + 1)] = 1.0
_SHB = np.zeros((4, 64), np.float32)
for _kk in range(4):
    _SHB[_kk, 16 * _kk:16 * (_kk + 1)] = 1.0


def _sc_gather(na, src):
    mesh = plsc.VectorSubcoreMesh(core_axis_name="c", subcore_axis_name="s")

    @functools.partial(
        pl.kernel,
        out_type=jax.ShapeDtypeStruct((_EH, _W), jnp.float32),
        mesh=mesh,
        scratch_types=[
            pltpu.VMEM((_G8,), jnp.int32),
            pltpu.VMEM((512, _W), jnp.float32),
            pltpu.SemaphoreType.DMA,
        ],
    )
    def gk(na_hbm, src_hbm, x_hbm, idx_v, rows_v, sem):
        c = lax.axis_index("c")
        s = lax.axis_index("s")
        wid = s * _NC + c

        def group(gid):
            base = pl.multiple_of(gid * _G8, _G8)
            pltpu.sync_copy(src_hbm.at[pl.ds(base, _G8)], idx_v)
            for half in range(2):
                descs = [pltpu.async_copy(
                    na_hbm.at[idx_v.at[pl.ds(half * 512 + _GC * i, _GC)]],
                    rows_v.at[pl.ds(_GC * i, _GC)], sem) for i in range(4)]
                for d in descs:
                    d.wait()
                pltpu.sync_copy(rows_v, x_hbm.at[pl.ds(base + half * 512, 512)])

        def body(j, carry):
            group(j * _NW + wid)
            return carry

        lax.fori_loop(0, _JF, body, 0)

        @pl.when(wid < _GEXT)
        def _():
            group(_JF * _NW + wid)

        @pl.when(wid == _GEXT)
        def _():
            pltpu.sync_copy(src_hbm.at[pl.ds(_REMB, _REMC)], idx_v.at[pl.ds(0, _REMC)])
            descs = [pltpu.async_copy(
                na_hbm.at[idx_v.at[pl.ds(_GC * i, _GC)]],
                rows_v.at[pl.ds(_GC * i, _GC)], sem) for i in range(_REMC // _GC)]
            for d in descs:
                d.wait()
            pltpu.sync_copy(rows_v.at[pl.ds(0, _REMC)], x_hbm.at[pl.ds(_REMB, _REMC)])

    return gk(na, src)


def _tc_dense(x, easb, W1, b1, Wc, Rep, Shb):
    def body(x_ref, easb_ref, w1_ref, b1_ref, wc_ref, rep_ref,
             shb_ref, o_ref):
        xb = x_ref[...]
        easbb = easb_ref[...]
        a = jnp.dot(easbb[:, 0:16], w1_ref[...],
                    preferred_element_type=jnp.float32) + b1_ref[...]
        h = a * (1.0 / (1.0 + jnp.exp(-a)))
        shb = jnp.dot(easbb[:, 16:20], shb_ref[...], preferred_element_type=jnp.float32)
        y0 = shb[:, 0:16]
        yv0 = shb[:, 16:32]
        yv1 = shb[:, 32:48]
        yv2 = shb[:, 48:64]
        xs = xb[:, 0:16]
        xv0 = xb[:, 16:32]
        xv1 = xb[:, 32:48]
        xv2 = xb[:, 48:64]
        zA = xs * y0
        dot = xv0 * yv0 + xv1 * yv1 + xv2 * yv2
        g = jnp.concatenate(
            [zA, dot, xs, xv0, xv1, xv2, jnp.zeros((_EB, 32), jnp.float32)], axis=1)
        Q = jnp.dot(g.astype(jnp.bfloat16), wc_ref[...],
                    preferred_element_type=jnp.float32)
        H = jnp.dot(h.astype(jnp.bfloat16), rep_ref[...],
                    preferred_element_type=jnp.float32)
        S = Q[:, 16 * 128:17 * 128]
        for k in range(16):
            S = S + H[:, 128 * k:128 * (k + 1)] * Q[:, 128 * k:128 * (k + 1)]
        out0 = _INV * S[:, 0:16]
        t = S[:, 16:32]
        o1 = [
            _INV * (t * shb[:, 16 + 16 * k:32 + 16 * k] + y0 * S[:, 32 + 16 * k:48 + 16 * k])
            for k in range(3)
        ]
        o_ref[...] = jnp.concatenate(
            [out0] + o1
            + [jnp.ones((_EB, 16), jnp.float32), jnp.zeros((_EB, 48), jnp.float32)],
            axis=1)

    return pl.pallas_call(
        body,
        grid=(_EH // _EB,),
        in_specs=[
            pl.BlockSpec((_EB, _W), lambda i: (i, 0)),
            pl.BlockSpec((_EB, 20), lambda i: (i, 0)),
            pl.BlockSpec((16, 16), lambda i: (0, 0)),
            pl.BlockSpec((1, 16), lambda i: (0, 0)),
            pl.BlockSpec((128, 17 * 128), lambda i: (0, 0)),
            pl.BlockSpec((16, 16 * 128), lambda i: (0, 0)),
            pl.BlockSpec((4, 64), lambda i: (0, 0)),
        ],
        out_specs=pl.BlockSpec((_EB, _W), lambda i: (i, 0)),
        out_shape=jax.ShapeDtypeStruct((_EH, _W), jnp.float32),
    )(x, easb, W1, b1, Wc, Rep, Shb)


def _sc_scatter(y, dst2p, zrow):
    mesh = plsc.VectorSubcoreMesh(core_axis_name="c", subcore_axis_name="s")

    @functools.partial(
        pl.kernel,
        out_type=jax.ShapeDtypeStruct((_NC, _N, _W), jnp.float32),
        mesh=mesh,
        scratch_types=[
            pltpu.VMEM((8, _GC), jnp.int32),
            pltpu.VMEM((_GC, _W), jnp.float32),
            pltpu.VMEM((_GC, _W), jnp.float32),
            pltpu.VMEM_SHARED((_N, _W), jnp.float32),
            pltpu.SemaphoreType.DMA,
            pltpu.SemaphoreType.DMA,
        ],
    )
    def sk(y_hbm, dst2_hbm, z_hbm, psum_hbm, idx_v, y_v, y_w, acc_sh, semy, sems):
        c = lax.axis_index("c")
        s = lax.axis_index("s")
        wid = s * _NC + c

        @pl.when(s == 0)
        def _():
            pltpu.sync_copy(z_hbm, acc_sh)

        plsc.subcore_barrier()

        bufs = None

        def group(gid):
            base = pl.multiple_of(gid * _G8, _G8)
            grow = pl.multiple_of(gid * 8, 8)
            pltpu.sync_copy(dst2_hbm.at[pl.ds(grow, 8)], idx_v)
            dl = {}
            ds_ = {}
            for i in range(8):
                buf = (y_v, y_w)[i % 2]
                if i >= 2:
                    ds_[i - 2].wait()
                dl[i] = pltpu.async_copy(
                    y_hbm.at[pl.ds(base + _GC * i, _GC)], buf, semy)
                if i >= 1:
                    bufj = (y_v, y_w)[(i - 1) % 2]
                    dl[i - 1].wait()
                    ds_[i - 1] = pltpu.async_copy(
                        bufj, acc_sh.at[idx_v.at[i - 1]], sems, add=True)
            dl[7].wait()
            ds_[7] = pltpu.async_copy(y_w, acc_sh.at[idx_v.at[7]], sems, add=True)
            ds_[6].wait()
            ds_[7].wait()

        def body(j, carry):
            group(j * _NW + wid)
            return carry

        lax.fori_loop(0, _JF, body, 0)

        @pl.when(wid < _GEXT)
        def _():
            group(_JF * _NW + wid)

        @pl.when(wid == _GEXT)
        def _():
            pltpu.sync_copy(dst2_hbm.at[pl.ds(_NGF * 8, 8)], idx_v)
            pltpu.sync_copy(y_hbm.at[pl.ds(_REMB, _GC)], y_v)
            pltpu.async_copy(y_v, acc_sh.at[idx_v.at[0]], sems, add=True).wait()

        plsc.subcore_barrier()

        nfull = _N // _GC
        rem = _N - nfull * _GC

        def dump(r0, nrows):
            pltpu.sync_copy(acc_sh.at[pl.ds(r0, nrows)], y_v.at[pl.ds(0, nrows)])
            pltpu.sync_copy(y_v.at[pl.ds(0, nrows)], psum_hbm.at[c, pl.ds(r0, nrows)])

        def dbody(j, carry):
            g = j * _NS + s

            @pl.when(g < nfull)
            def _():
                dump(pl.multiple_of(g * _GC, _GC), _GC)

            @pl.when(g == nfull)
            def _():
                dump(nfull * _GC, rem)

            return carry

        lax.fori_loop(0, (nfull + _NS) // _NS, dbody, 0)

    return sk(y, dst2p, zrow)


_CPM = np.zeros((64, 64), np.float32)
for _j in range(64):
    _CPM[_COLMAP[_j], _j] = 1.0


def _tc_finalize(psum1, psum2):
    def body(p1_ref, p2_ref, pm_ref, o_ref):
        ssum = (p1_ref[0, :, 0:64] + p1_ref[1, :, 0:64]
                + p2_ref[0, :, 0:64] + p2_ref[1, :, 0:64])
        cnt = (p1_ref[0, :, 64:65] + p1_ref[1, :, 64:65]
               + p2_ref[0, :, 64:65] + p2_ref[1, :, 64:65])
        o_ref[...] = jnp.dot(ssum / jnp.maximum(cnt, 1.0), pm_ref[...],
                             preferred_element_type=jnp.float32)

    return pl.pallas_call(
        body,
        out_shape=jax.ShapeDtypeStruct((_N, 64), jnp.float32),
    )(psum1, psum2, jnp.asarray(_CPM))


def kernel(node_attr, edge_index, edge_attr, edge_sh, W1, b1, W2, b2):
    na = jnp.pad(node_attr[:, _PERM], ((0, 0), (0, _W - 64)))
    src = edge_index[0]
    dst = edge_index[1]
    Wc = _assemble_wc(W2, b2).astype(jnp.bfloat16)
    Rep = jnp.asarray(_REP, jnp.bfloat16)
    Shb = jnp.asarray(_SHB)
    b1r = b1.reshape(1, 16)
    zrow = jnp.zeros((_N, _W), jnp.float32)

    def dst2pad(d1):
        return jnp.pad(d1.reshape(_EH // _GC, _GC), ((0, 7), (0, 0)))

    easb = jnp.concatenate([edge_attr, edge_sh], axis=1)
    x1 = _sc_gather(na, src[:_EH])
    x2 = _sc_gather(na, src[_EH:])
    y1 = _tc_dense(x1, easb[:_EH], W1, b1r, Wc, Rep, Shb)
    psum1 = _sc_scatter(y1, dst2pad(dst[:_EH]), zrow)
    y2 = _tc_dense(x2, easb[_EH:], W1, b1r, Wc, Rep, Shb)
    psum2 = _sc_scatter(y2, dst2pad(dst[_EH:]), zrow)
    return _tc_finalize(psum1, psum2)
